# Initial kernel scaffold; baseline (speedup 1.0000x reference)
import jax
import jax.numpy as jnp
from jax.experimental import pallas as pl


def kernel(nodes, edges, senders, receivers, globals_, We1, be1, We2, be2, Wn1, bn1, Wn2, bn2, Wg1, bg1, Wg2, bg2):
    raise NotImplementedError("write your pallas kernel here")


# speedup vs baseline: 7.0187x; 7.0187x over previous
import functools

import jax
import jax.numpy as jnp
from jax import lax
from jax.experimental import pallas as pl
from jax.experimental.pallas import tpu as pltpu
from jax.experimental.pallas import tpu_sc as plsc

_HIGH = lax.Precision.HIGHEST


def _dot(a, b):
    return lax.dot_general(a, b, (((a.ndim - 1,), (0,)), ((), ())),
                           precision=_HIGH, preferred_element_type=jnp.float32)


def _node_proj(nodes, Ws, Wr):
    N = nodes.shape[0]
    L = Ws.shape[1]

    def body(n_ref, ws_ref, wr_ref, os_ref, or_ref):
        x = n_ref[...]
        os_ref[...] = _dot(x, ws_ref[...])
        or_ref[...] = _dot(x, wr_ref[...])

    return pl.pallas_call(
        body,
        out_shape=[jax.ShapeDtypeStruct((N, L), jnp.float32),
                   jax.ShapeDtypeStruct((N, L), jnp.float32)],
    )(nodes, Ws, Wr)


def _edge_linear(e2, Wbd, g, Wg, b1, KT):
    E8 = e2.shape[0]
    BB = 5000

    def body(e_ref, w_ref, g_ref, wg_ref, b_ref, kt_ref, o_ref):
        c16 = _dot(g_ref[...], wg_ref[...]) + b_ref[...]
        o_ref[...] = _dot(e_ref[...], w_ref[...]) + _dot(c16, kt_ref[...])

    return pl.pallas_call(
        body, grid=(E8 // BB,),
        in_specs=[pl.BlockSpec((BB, 128), lambda i: (i, 0)),
                  pl.BlockSpec((128, 128), lambda i: (0, 0)),
                  pl.BlockSpec((1, 16), lambda i: (0, 0)),
                  pl.BlockSpec((16, 16), lambda i: (0, 0)),
                  pl.BlockSpec((1, 16), lambda i: (0, 0)),
                  pl.BlockSpec((16, 128), lambda i: (0, 0))],
        out_specs=pl.BlockSpec((BB, 128), lambda i: (i, 0)),
        out_shape=jax.ShapeDtypeStruct((E8, 128), jnp.float32),
    )(e2, Wbd, g, Wg, b1, KT)


def _edge_out(h2, Wbd, b128):
    E8 = h2.shape[0]
    BB = 5000

    def body(h_ref, w_ref, b_ref, o_ref, s_ref):
        i = pl.program_id(0)
        h = h_ref[...]
        o_ref[...] = _dot(h, w_ref[...]) + b_ref[...]

        @pl.when(i == 0)
        def _():
            s_ref[...] = jnp.zeros_like(s_ref)

        s_ref[...] += jnp.sum(h, axis=0, keepdims=True)

    return pl.pallas_call(
        body, grid=(E8 // BB,),
        in_specs=[pl.BlockSpec((BB, 128), lambda i: (i, 0)),
                  pl.BlockSpec((128, 128), lambda i: (0, 0)),
                  pl.BlockSpec((1, 128), lambda i: (0, 0))],
        out_specs=[pl.BlockSpec((BB, 128), lambda i: (i, 0)),
                   pl.BlockSpec((1, 128), lambda i: (0, 0))],
        out_shape=[jax.ShapeDtypeStruct((E8, 128), jnp.float32),
                   jax.ShapeDtypeStruct((1, 128), jnp.float32)],
    )(h2, Wbd, b128)


def _node_block(nodes, ssum, srum, cs16, cr16, g, W1n, W1s, W1r, W1g, b1,
                W2, b2, We2, be2):
    N, DF = nodes.shape
    L = W2.shape[0]
    BN = 2000
    NB = N // BN

    def body(n_ref, ss0, ss1, sr0, sr1, c0s, c1s, c0r, c1r, g_ref,
             w1n, w1s, w1r, w1g, b1r, w2r, b2r, we2r, be2r, o_ref):
        cnt_s = c0s[...] + c1s[...]
        cnt_r = c0r[...] + c1r[...]
        ms = (ss0[...] + ss1[...]) / jnp.maximum(cnt_s, 1.0)
        mr = (sr0[...] + sr1[...]) / jnp.maximum(cnt_r, 1.0)
        sent_agg = _dot(ms, we2r[...]) + jnp.minimum(cnt_s, 1.0) * be2r[...]
        recv_agg = _dot(mr, we2r[...]) + jnp.minimum(cnt_r, 1.0) * be2r[...]
        pre = (_dot(n_ref[...], w1n[...]) + _dot(sent_agg, w1s[...])
               + _dot(recv_agg, w1r[...]) + _dot(g_ref[...], w1g[...])
               + b1r[...])
        o_ref[...] = _dot(jnp.maximum(pre, 0.0), w2r[...]) + b2r[...]

    half = [pl.BlockSpec((BN, L), lambda i: (i, 0)),
            pl.BlockSpec((BN, L), lambda i: (i + NB, 0))]
    return pl.pallas_call(
        body, grid=(NB,),
        in_specs=[pl.BlockSpec((BN, DF), lambda i: (i, 0))]
        + half + half + half + half
        + [pl.BlockSpec((1, 16), lambda i: (0, 0)),
           pl.BlockSpec((DF, L), lambda i: (0, 0))]
        + [pl.BlockSpec((L, L), lambda i: (0, 0))] * 2
        + [pl.BlockSpec((16, L), lambda i: (0, 0)),
           pl.BlockSpec((1, L), lambda i: (0, 0)),
           pl.BlockSpec((L, L), lambda i: (0, 0)),
           pl.BlockSpec((1, L), lambda i: (0, 0)),
           pl.BlockSpec((L, L), lambda i: (0, 0)),
           pl.BlockSpec((1, L), lambda i: (0, 0))],
        out_specs=pl.BlockSpec((BN, L), lambda i: (i, 0)),
        out_shape=jax.ShapeDtypeStruct((N, L), jnp.float32),
    )(nodes, ssum, ssum, srum, srum, cs16, cs16, cr16, cr16, g,
      W1n, W1s, W1r, W1g, b1, W2, b2, We2, be2)


def _glob_block(new_nodes, es128, g, S, We2, be2, Wg1n, Wg1e, Wg1g, bg1,
                Wg2, bg2, n_edge):
    N, L = new_nodes.shape

    def body(nn_ref, es_ref, g_ref, s_ref, we2r, be2r,
             wgn, wge, wgg, bg1r, wg2r, bg2r, o_ref):
        node_agg = jnp.sum(nn_ref[...], axis=0, keepdims=True) * (1.0 / N)
        h1sum = _dot(es_ref[...], s_ref[...]) * (1.0 / n_edge)
        edge_agg = _dot(h1sum, we2r[...]) + be2r[...]
        pre = (_dot(node_agg, wgn[...]) + _dot(edge_agg, wge[...])
               + _dot(g_ref[...], wgg[...]) + bg1r[...])
        o_ref[...] = _dot(jnp.maximum(pre, 0.0), wg2r[...]) + bg2r[...]

    return pl.pallas_call(
        body,
        out_shape=jax.ShapeDtypeStruct((1, L), jnp.float32),
    )(new_nodes, es128, g, S, We2, be2, Wg1n, Wg1e, Wg1g, bg1, Wg2, bg2)


def _make_sc_edge(N, E, L):
    info = plsc.get_sparse_core_info()
    NC, NS = info.num_cores, info.num_subcores
    NW = NC * NS
    EW = E // NW
    C = 1000
    NCH = EW // C
    CH = 1000
    NJ = N // CH
    mesh = plsc.VectorSubcoreMesh(core_axis_name="c", subcore_axis_name="s")

    @functools.partial(
        pl.kernel,
        mesh=mesh,
        compiler_params=pltpu.CompilerParams(use_tc_tiling_on_sc=False),
        out_type=[
            jax.ShapeDtypeStruct((E, L), jnp.float32),
            jax.ShapeDtypeStruct((NC * N, L), jnp.float32),
            jax.ShapeDtypeStruct((NC * N, L), jnp.float32),
            jax.ShapeDtypeStruct((NC * N, L), jnp.float32),
            jax.ShapeDtypeStruct((NC * N, L), jnp.float32),
        ],
        scratch_types=[
            pltpu.VMEM((C,), jnp.int32),
            pltpu.VMEM((C,), jnp.int32),
            pltpu.VMEM((C, L), jnp.float32),
            pltpu.VMEM((C, L), jnp.float32),
            pltpu.VMEM((C, L), jnp.float32),
            pltpu.VMEM((C, L), jnp.float32),
            pltpu.VMEM_SHARED((N, L), jnp.float32),
            pltpu.VMEM_SHARED((N, L), jnp.float32),
            pltpu.VMEM_SHARED((N, L), jnp.float32),
            pltpu.VMEM_SHARED((N, L), jnp.float32),
            pltpu.SemaphoreType.DMA,
            pltpu.SemaphoreType.DMA,
        ],
    )
    def sc_edge(ps_hbm, pr_hbm, elin_hbm, s_hbm, r_hbm,
                h1_out, ss_out, sr_out, cs_out, cr_out,
                idx_s, idx_r, buf_s, buf_r, buf_e, ones_v,
                acc_s, acc_r, acc_cs, acc_cr, sem1, sem2):
        cid = lax.axis_index("c")
        sid = lax.axis_index("s")
        wid = cid * NS + sid

        zero16 = jnp.zeros((16,), jnp.float32)
        one16 = jnp.full((16,), 1.0, jnp.float32)

        def fill(i, _):
            buf_s[i] = zero16
            ones_v[i] = one16
            return 0
        lax.fori_loop(0, C, fill, 0)

        accs = [acc_s, acc_r, acc_cs, acc_cr]
        for a, acc in enumerate(accs):
            for c in range(NJ):
                @pl.when(sid == (a * NJ + c) % NS)
                def _():
                    pltpu.sync_copy(buf_s, acc.at[pl.ds(c * CH, CH)])
        plsc.subcore_barrier()

        ebase = wid * EW
        for k in range(NCH):
            off = ebase + k * C
            pltpu.sync_copy(s_hbm.at[pl.ds(off, C)], idx_s)
            pltpu.sync_copy(r_hbm.at[pl.ds(off, C)], idx_r)
            cp1 = pltpu.async_copy(ps_hbm.at[idx_s], buf_s, sem1)
            cp2 = pltpu.async_copy(pr_hbm.at[idx_r], buf_r, sem2)
            pltpu.sync_copy(elin_hbm.at[pl.ds(off, C)], buf_e)
            cp1.wait()
            cp2.wait()

            def addrelu(i, _):
                buf_e[i] = jnp.maximum(buf_e[i] + buf_s[i] + buf_r[i], 0.0)
                return 0
            lax.fori_loop(0, C, addrelu, 0)

            pltpu.sync_copy(buf_e, h1_out.at[pl.ds(off, C)])
            pltpu.sync_copy(buf_e, acc_s.at[idx_s], add=True)
            pltpu.sync_copy(buf_e, acc_r.at[idx_r], add=True)
            pltpu.sync_copy(ones_v, acc_cs.at[idx_s], add=True)
            pltpu.sync_copy(ones_v, acc_cr.at[idx_r], add=True)

        plsc.subcore_barrier()
        outs = [ss_out, sr_out, cs_out, cr_out]
        for a in range(4):
            for c in range(NJ):
                @pl.when(sid == (a * NJ + c) % NS)
                def _(a=a, c=c):
                    pltpu.sync_copy(accs[a].at[pl.ds(c * CH, CH)],
                                    outs[a].at[pl.ds(cid * N + c * CH, CH)])

    return sc_edge


def kernel(nodes, edges, senders, receivers, globals_, We1, be1, We2, be2,
           Wn1, bn1, Wn2, bn2, Wg1, bg1, Wg2, bg2):
    N, DF = nodes.shape
    E, DE = edges.shape
    L = We2.shape[0]
    DG = globals_.shape[1]

    We1_e = We1[:DE]
    We1_s = We1[DE:DE + DF]
    We1_r = We1[DE + DF:DE + 2 * DF]
    We1_g = We1[DE + 2 * DF:]
    Wn1_n = Wn1[:DF]
    Wn1_s = Wn1[DF:DF + L]
    Wn1_r = Wn1[DF + L:DF + 2 * L]
    Wn1_g = Wn1[DF + 2 * L:]
    Wg1_n = Wg1[:L]
    Wg1_e = Wg1[L:2 * L]
    Wg1_g = Wg1[2 * L:]

    eye8 = jnp.eye(8, dtype=jnp.float32)
    WeBD = jnp.kron(eye8, We1_e)
    W2BD = jnp.kron(eye8, We2)
    KT = jnp.kron(jnp.ones((1, 8), jnp.float32), jnp.eye(L, dtype=jnp.float32))
    S = jnp.kron(jnp.ones((8, 1), jnp.float32), jnp.eye(L, dtype=jnp.float32))
    b1r = be1.reshape(1, L)
    b2_128 = jnp.tile(be2.reshape(1, L), (1, 8))
    be2r = be2.reshape(1, L)

    ps, pr = _node_proj(nodes, We1_s, We1_r)
    elin2 = _edge_linear(edges.reshape(E // 8, 8 * DE), WeBD, globals_,
                         We1_g, b1r, KT)

    sc_edge = _make_sc_edge(N, E, L)
    h1, ssum, srum, cs16, cr16 = sc_edge(
        ps, pr, elin2.reshape(E, L), senders, receivers)

    ne2, es128 = _edge_out(h1.reshape(E // 8, 8 * L), W2BD, b2_128)
    new_edges = ne2.reshape(E, L)

    new_nodes = _node_block(nodes, ssum, srum, cs16, cr16, globals_,
                            Wn1_n, Wn1_s, Wn1_r, Wn1_g, bn1.reshape(1, L),
                            Wn2, bn2.reshape(1, L), We2, be2r)
    new_globals = _glob_block(new_nodes, es128, globals_, S, We2, be2r,
                              Wg1_n, Wg1_e, Wg1_g, bg1.reshape(1, L),
                              Wg2, bg2.reshape(1, L), E)
    return new_nodes, new_edges, new_globals


# speedup vs baseline: 7.6786x; 1.0940x over previous
import functools

import jax
import jax.numpy as jnp
from jax import lax
from jax.experimental import pallas as pl
from jax.experimental.pallas import tpu as pltpu
from jax.experimental.pallas import tpu_sc as plsc

_HIGH = lax.Precision.HIGHEST


def _dot(a, b):
    return lax.dot_general(a, b, (((a.ndim - 1,), (0,)), ((), ())),
                           precision=_HIGH, preferred_element_type=jnp.float32)


def _node_proj(nodes, Ws, Wr):
    N = nodes.shape[0]
    L = Ws.shape[1]

    def body(n_ref, ws_ref, wr_ref, os_ref, or_ref):
        x = n_ref[...]
        os_ref[...] = _dot(x, ws_ref[...])
        or_ref[...] = _dot(x, wr_ref[...])

    return pl.pallas_call(
        body,
        out_shape=[jax.ShapeDtypeStruct((N, L), jnp.float32),
                   jax.ShapeDtypeStruct((N, L), jnp.float32)],
    )(nodes, Ws, Wr)


def _edge_linear(e2, Wbd, g, Wg, b1, KT):
    E8 = e2.shape[0]
    BB = 5000

    def body(e_ref, w_ref, g_ref, wg_ref, b_ref, kt_ref, o_ref):
        c16 = _dot(g_ref[...], wg_ref[...]) + b_ref[...]
        o_ref[...] = _dot(e_ref[...], w_ref[...]) + _dot(c16, kt_ref[...])

    return pl.pallas_call(
        body, grid=(E8 // BB,),
        in_specs=[pl.BlockSpec((BB, 128), lambda i: (i, 0)),
                  pl.BlockSpec((128, 128), lambda i: (0, 0)),
                  pl.BlockSpec((1, 16), lambda i: (0, 0)),
                  pl.BlockSpec((16, 16), lambda i: (0, 0)),
                  pl.BlockSpec((1, 16), lambda i: (0, 0)),
                  pl.BlockSpec((16, 128), lambda i: (0, 0))],
        out_specs=pl.BlockSpec((BB, 128), lambda i: (i, 0)),
        out_shape=jax.ShapeDtypeStruct((E8, 128), jnp.float32),
    )(e2, Wbd, g, Wg, b1, KT)


def _edge_out(h2, Wbd, b128):
    E8 = h2.shape[0]
    BB = 5000

    def body(h_ref, w_ref, b_ref, o_ref, s_ref):
        i = pl.program_id(0)
        h = h_ref[...]
        o_ref[...] = _dot(h, w_ref[...]) + b_ref[...]

        @pl.when(i == 0)
        def _():
            s_ref[...] = jnp.zeros_like(s_ref)

        s_ref[...] += jnp.sum(h, axis=0, keepdims=True)

    return pl.pallas_call(
        body, grid=(E8 // BB,),
        in_specs=[pl.BlockSpec((BB, 128), lambda i: (i, 0)),
                  pl.BlockSpec((128, 128), lambda i: (0, 0)),
                  pl.BlockSpec((1, 128), lambda i: (0, 0))],
        out_specs=[pl.BlockSpec((BB, 128), lambda i: (i, 0)),
                   pl.BlockSpec((1, 128), lambda i: (0, 0))],
        out_shape=[jax.ShapeDtypeStruct((E8, 128), jnp.float32),
                   jax.ShapeDtypeStruct((1, 128), jnp.float32)],
    )(h2, Wbd, b128)


def _node_block(nodes, ssum, srum, cs16, cr16, g, W1n, W1s, W1r, W1g, b1,
                W2, b2, We2, be2):
    N, DF = nodes.shape
    L = W2.shape[0]
    BN = 2000
    NB = N // BN

    def body(n_ref, ss0, ss1, sr0, sr1, c0s, c1s, c0r, c1r, g_ref,
             w1n, w1s, w1r, w1g, b1r, w2r, b2r, we2r, be2r, o_ref):
        cnt_s = c0s[...] + c1s[...]
        cnt_r = c0r[...] + c1r[...]
        ms = (ss0[...] + ss1[...]) / jnp.maximum(cnt_s, 1.0)
        mr = (sr0[...] + sr1[...]) / jnp.maximum(cnt_r, 1.0)
        sent_agg = _dot(ms, we2r[...]) + jnp.minimum(cnt_s, 1.0) * be2r[...]
        recv_agg = _dot(mr, we2r[...]) + jnp.minimum(cnt_r, 1.0) * be2r[...]
        pre = (_dot(n_ref[...], w1n[...]) + _dot(sent_agg, w1s[...])
               + _dot(recv_agg, w1r[...]) + _dot(g_ref[...], w1g[...])
               + b1r[...])
        o_ref[...] = _dot(jnp.maximum(pre, 0.0), w2r[...]) + b2r[...]

    half = [pl.BlockSpec((BN, L), lambda i: (i, 0)),
            pl.BlockSpec((BN, L), lambda i: (i + NB, 0))]
    return pl.pallas_call(
        body, grid=(NB,),
        in_specs=[pl.BlockSpec((BN, DF), lambda i: (i, 0))]
        + half + half + half + half
        + [pl.BlockSpec((1, 16), lambda i: (0, 0)),
           pl.BlockSpec((DF, L), lambda i: (0, 0))]
        + [pl.BlockSpec((L, L), lambda i: (0, 0))] * 2
        + [pl.BlockSpec((16, L), lambda i: (0, 0)),
           pl.BlockSpec((1, L), lambda i: (0, 0)),
           pl.BlockSpec((L, L), lambda i: (0, 0)),
           pl.BlockSpec((1, L), lambda i: (0, 0)),
           pl.BlockSpec((L, L), lambda i: (0, 0)),
           pl.BlockSpec((1, L), lambda i: (0, 0))],
        out_specs=pl.BlockSpec((BN, L), lambda i: (i, 0)),
        out_shape=jax.ShapeDtypeStruct((N, L), jnp.float32),
    )(nodes, ssum, ssum, srum, srum, cs16, cs16, cr16, cr16, g,
      W1n, W1s, W1r, W1g, b1, W2, b2, We2, be2)


def _glob_block(new_nodes, es128, g, S, We2, be2, Wg1n, Wg1e, Wg1g, bg1,
                Wg2, bg2, n_edge):
    N, L = new_nodes.shape

    def body(nn_ref, es_ref, g_ref, s_ref, we2r, be2r,
             wgn, wge, wgg, bg1r, wg2r, bg2r, o_ref):
        node_agg = jnp.sum(nn_ref[...], axis=0, keepdims=True) * (1.0 / N)
        h1sum = _dot(es_ref[...], s_ref[...]) * (1.0 / n_edge)
        edge_agg = _dot(h1sum, we2r[...]) + be2r[...]
        pre = (_dot(node_agg, wgn[...]) + _dot(edge_agg, wge[...])
               + _dot(g_ref[...], wgg[...]) + bg1r[...])
        o_ref[...] = _dot(jnp.maximum(pre, 0.0), wg2r[...]) + bg2r[...]

    return pl.pallas_call(
        body,
        out_shape=jax.ShapeDtypeStruct((1, L), jnp.float32),
    )(new_nodes, es128, g, S, We2, be2, Wg1n, Wg1e, Wg1g, bg1, Wg2, bg2)


def _make_sc_edge(N, E, L):
    info = plsc.get_sparse_core_info()
    NC, NS = info.num_cores, info.num_subcores
    NW = NC * NS
    EW = E // NW
    C = 1000
    NCH = EW // C
    CH = 1000
    NJ = N // CH
    mesh = plsc.VectorSubcoreMesh(core_axis_name="c", subcore_axis_name="s")

    @functools.partial(
        pl.kernel,
        mesh=mesh,
        compiler_params=pltpu.CompilerParams(use_tc_tiling_on_sc=False),
        out_type=[
            jax.ShapeDtypeStruct((E // 8, 8 * L), jnp.float32),
            jax.ShapeDtypeStruct((NC * N, L), jnp.float32),
            jax.ShapeDtypeStruct((NC * N, L), jnp.float32),
            jax.ShapeDtypeStruct((NC * N, L), jnp.float32),
            jax.ShapeDtypeStruct((NC * N, L), jnp.float32),
        ],
        scratch_types=[
            pltpu.VMEM((C,), jnp.int32),
            pltpu.VMEM((C,), jnp.int32),
            pltpu.VMEM((C, L), jnp.float32),
            pltpu.VMEM((C, L), jnp.float32),
            pltpu.VMEM((C // 8, 8 * L), jnp.float32),
            pltpu.VMEM((C, L), jnp.float32),
            pltpu.VMEM_SHARED((N, L), jnp.float32),
            pltpu.VMEM_SHARED((N, L), jnp.float32),
            pltpu.VMEM_SHARED((N, L), jnp.float32),
            pltpu.VMEM_SHARED((N, L), jnp.float32),
            pltpu.SemaphoreType.DMA,
            pltpu.SemaphoreType.DMA,
        ],
    )
    def sc_edge(ps_hbm, pr_hbm, elin_hbm, s_hbm, r_hbm,
                h1_out, ss_out, sr_out, cs_out, cr_out,
                idx_s, idx_r, buf_s, buf_r, buf_e, ones_v,
                acc_s, acc_r, acc_cs, acc_cr, sem1, sem2):
        cid = lax.axis_index("c")
        sid = lax.axis_index("s")
        wid = cid * NS + sid

        zero16 = jnp.zeros((16,), jnp.float32)
        one16 = jnp.full((16,), 1.0, jnp.float32)

        def fill(i, _):
            buf_s[i] = zero16
            ones_v[i] = one16
            return 0
        lax.fori_loop(0, C, fill, 0)

        accs = [acc_s, acc_r, acc_cs, acc_cr]
        for a, acc in enumerate(accs):
            for c in range(NJ):
                @pl.when(sid == (a * NJ + c) % NS)
                def _():
                    pltpu.sync_copy(buf_s, acc.at[pl.ds(c * CH, CH)])
        plsc.subcore_barrier()

        ebase = wid * EW
        RB = C // 8
        for k in range(NCH):
            off = ebase + k * C
            off8 = (ebase + k * C) // 8
            pltpu.sync_copy(s_hbm.at[pl.ds(off, C)], idx_s)
            pltpu.sync_copy(r_hbm.at[pl.ds(off, C)], idx_r)
            cp1 = pltpu.async_copy(ps_hbm.at[idx_s], buf_s, sem1)
            cp2 = pltpu.async_copy(pr_hbm.at[idx_r], buf_r, sem2)
            pltpu.sync_copy(elin_hbm.at[pl.ds(off8, RB)], buf_e)
            cp1.wait()
            cp2.wait()

            def addrelu(j, _):
                for m in range(8):
                    i = j * 8 + m
                    v = jnp.maximum(
                        buf_e[j, pl.ds(m * L, L)] + buf_s[i] + buf_r[i], 0.0)
                    buf_e[j, pl.ds(m * L, L)] = v
                    buf_s[i] = v
                return 0
            lax.fori_loop(0, RB, addrelu, 0)

            pltpu.sync_copy(buf_e, h1_out.at[pl.ds(off8, RB)])
            pltpu.sync_copy(buf_s, acc_s.at[idx_s], add=True)
            pltpu.sync_copy(buf_s, acc_r.at[idx_r], add=True)
            pltpu.sync_copy(ones_v, acc_cs.at[idx_s], add=True)
            pltpu.sync_copy(ones_v, acc_cr.at[idx_r], add=True)

        plsc.subcore_barrier()
        outs = [ss_out, sr_out, cs_out, cr_out]
        for a in range(4):
            for c in range(NJ):
                @pl.when(sid == (a * NJ + c) % NS)
                def _(a=a, c=c):
                    pltpu.sync_copy(accs[a].at[pl.ds(c * CH, CH)],
                                    outs[a].at[pl.ds(cid * N + c * CH, CH)])

    return sc_edge


def kernel(nodes, edges, senders, receivers, globals_, We1, be1, We2, be2,
           Wn1, bn1, Wn2, bn2, Wg1, bg1, Wg2, bg2):
    N, DF = nodes.shape
    E, DE = edges.shape
    L = We2.shape[0]
    DG = globals_.shape[1]

    We1_e = We1[:DE]
    We1_s = We1[DE:DE + DF]
    We1_r = We1[DE + DF:DE + 2 * DF]
    We1_g = We1[DE + 2 * DF:]
    Wn1_n = Wn1[:DF]
    Wn1_s = Wn1[DF:DF + L]
    Wn1_r = Wn1[DF + L:DF + 2 * L]
    Wn1_g = Wn1[DF + 2 * L:]
    Wg1_n = Wg1[:L]
    Wg1_e = Wg1[L:2 * L]
    Wg1_g = Wg1[2 * L:]

    eye8 = jnp.eye(8, dtype=jnp.float32)
    WeBD = jnp.kron(eye8, We1_e)
    W2BD = jnp.kron(eye8, We2)
    KT = jnp.kron(jnp.ones((1, 8), jnp.float32), jnp.eye(L, dtype=jnp.float32))
    S = jnp.kron(jnp.ones((8, 1), jnp.float32), jnp.eye(L, dtype=jnp.float32))
    b1r = be1.reshape(1, L)
    b2_128 = jnp.tile(be2.reshape(1, L), (1, 8))
    be2r = be2.reshape(1, L)

    ps, pr = _node_proj(nodes, We1_s, We1_r)
    elin2 = _edge_linear(edges.reshape(E // 8, 8 * DE), WeBD, globals_,
                         We1_g, b1r, KT)

    sc_edge = _make_sc_edge(N, E, L)
    h1, ssum, srum, cs16, cr16 = sc_edge(ps, pr, elin2, senders, receivers)

    ne2, es128 = _edge_out(h1, W2BD, b2_128)
    new_edges = ne2.reshape(E, L)

    new_nodes = _node_block(nodes, ssum, srum, cs16, cr16, globals_,
                            Wn1_n, Wn1_s, Wn1_r, Wn1_g, bn1.reshape(1, L),
                            Wn2, bn2.reshape(1, L), We2, be2r)
    new_globals = _glob_block(new_nodes, es128, globals_, S, We2, be2r,
                              Wg1_n, Wg1_e, Wg1_g, bg1.reshape(1, L),
                              Wg2, bg2.reshape(1, L), E)
    return new_nodes, new_edges, new_globals


# speedup vs baseline: 8.4391x; 1.0990x over previous
import functools

import jax
import jax.numpy as jnp
from jax import lax
from jax.experimental import pallas as pl
from jax.experimental.pallas import tpu as pltpu
from jax.experimental.pallas import tpu_sc as plsc

_HIGH = lax.Precision.HIGHEST


def _dotg(a, b, dims):
    return lax.dot_general(a, b, (dims, ((), ())), precision=_HIGH,
                           preferred_element_type=jnp.float32)


def _dot(a, b):
    return _dotg(a, b, ((a.ndim - 1,), (0,)))


def _node_proj(nodes3, Ws, Wr):
    N8 = nodes3.shape[0]

    def body(n_ref, ws_ref, wr_ref, os_ref, or_ref):
        n3 = n_ref[...]
        ps, pr = [], []
        for k in range(8):
            nk = n3[:, k, :]
            ps.append(_dot(nk, ws_ref[...]))
            pr.append(_dot(nk, wr_ref[...]))
        os_ref[...] = jnp.concatenate(ps, axis=1)
        or_ref[...] = jnp.concatenate(pr, axis=1)

    return pl.pallas_call(
        body,
        out_shape=[jax.ShapeDtypeStruct((N8, 128), jnp.float32),
                   jax.ShapeDtypeStruct((N8, 128), jnp.float32)],
    )(nodes3, Ws, Wr)


def _edge_linear(e2, Wbd, g, Wg, b1, KT):
    E8 = e2.shape[0]
    BB = 5000

    def body(e_ref, w_ref, g_ref, wg_ref, b_ref, kt_ref, o_ref):
        c16 = _dot(g_ref[...], wg_ref[...]) + b_ref[...]
        o_ref[...] = _dot(e_ref[...], w_ref[...]) + _dot(c16, kt_ref[...])

    return pl.pallas_call(
        body, grid=(E8 // BB,),
        in_specs=[pl.BlockSpec((BB, 128), lambda i: (i, 0)),
                  pl.BlockSpec((128, 128), lambda i: (0, 0)),
                  pl.BlockSpec((1, 16), lambda i: (0, 0)),
                  pl.BlockSpec((16, 16), lambda i: (0, 0)),
                  pl.BlockSpec((1, 16), lambda i: (0, 0)),
                  pl.BlockSpec((16, 128), lambda i: (0, 0))],
        out_specs=pl.BlockSpec((BB, 128), lambda i: (i, 0)),
        out_shape=jax.ShapeDtypeStruct((E8, 128), jnp.float32),
    )(e2, Wbd, g, Wg, b1, KT)


def _edge_out(h2, Wbd, b128):
    E8 = h2.shape[0]
    BB = 5000

    def body(h_ref, w_ref, b_ref, o_ref, s_ref):
        i = pl.program_id(0)
        h = h_ref[...]
        o_ref[...] = _dot(h, w_ref[...]) + b_ref[...]

        @pl.when(i == 0)
        def _():
            s_ref[...] = jnp.zeros_like(s_ref)

        s_ref[...] += jnp.sum(h, axis=0, keepdims=True)

    return pl.pallas_call(
        body, grid=(E8 // BB,),
        in_specs=[pl.BlockSpec((BB, 128), lambda i: (i, 0)),
                  pl.BlockSpec((128, 128), lambda i: (0, 0)),
                  pl.BlockSpec((1, 128), lambda i: (0, 0))],
        out_specs=[pl.BlockSpec((BB, 128), lambda i: (i, 0)),
                   pl.BlockSpec((1, 128), lambda i: (0, 0))],
        out_shape=[jax.ShapeDtypeStruct((E8, 128), jnp.float32),
                   jax.ShapeDtypeStruct((1, 128), jnp.float32)],
    )(h2, Wbd, b128)


def _node_block(nodes3, ssum_b, srum_b, cs_b, cr_b, g, W1n, W1s_bd, W1r_bd,
                W1g, b1, W2_bd, b2_128, We2_bd, be2_128, KT):
    N8 = nodes3.shape[0]

    def body(n_ref, ss_ref, sr_ref, cs_ref, cr_ref, g_ref,
             w1n, w1s, w1r, w1g, b1r, w2r, b2r, we2r, be2r, kt_ref,
             o_ref, s_ref):
        ss = ss_ref[...]
        sr = sr_ref[...]
        cs = cs_ref[...]
        cr = cr_ref[...]
        cnt_s = cs[:N8] + cs[N8:]
        cnt_r = cr[:N8] + cr[N8:]
        ms = (ss[:N8] + ss[N8:]) / jnp.maximum(cnt_s, 1.0)
        mr = (sr[:N8] + sr[N8:]) / jnp.maximum(cnt_r, 1.0)
        sent_agg = _dot(ms, we2r[...]) + jnp.minimum(cnt_s, 1.0) * be2r[...]
        recv_agg = _dot(mr, we2r[...]) + jnp.minimum(cnt_r, 1.0) * be2r[...]
        n3 = n_ref[...]
        nparts = [_dot(n3[:, k, :], w1n[...]) for k in range(8)]
        gn = _dot(g_ref[...], w1g[...]) + b1r[...]
        pre = (jnp.concatenate(nparts, axis=1)
               + _dot(sent_agg, w1s[...]) + _dot(recv_agg, w1r[...])
               + _dot(gn, kt_ref[...]))
        nn = _dot(jnp.maximum(pre, 0.0), w2r[...]) + b2r[...]
        o_ref[...] = nn
        s_ref[...] = jnp.sum(nn, axis=0, keepdims=True)

    return pl.pallas_call(
        body,
        out_shape=[jax.ShapeDtypeStruct((N8, 128), jnp.float32),
                   jax.ShapeDtypeStruct((1, 128), jnp.float32)],
    )(nodes3, ssum_b, srum_b, cs_b, cr_b, g,
      W1n, W1s_bd, W1r_bd, W1g, b1, W2_bd, b2_128, We2_bd, be2_128, KT)


def _glob_block(nsum128, es128, g, S, We2, be2, Wg1n, Wg1e, Wg1g, bg1,
                Wg2, bg2, n_node, n_edge):
    def body(ns_ref, es_ref, g_ref, s_ref, we2r, be2r,
             wgn, wge, wgg, bg1r, wg2r, bg2r, o_ref):
        node_agg = _dot(ns_ref[...], s_ref[...]) * (1.0 / n_node)
        h1sum = _dot(es_ref[...], s_ref[...]) * (1.0 / n_edge)
        edge_agg = _dot(h1sum, we2r[...]) + be2r[...]
        pre = (_dot(node_agg, wgn[...]) + _dot(edge_agg, wge[...])
               + _dot(g_ref[...], wgg[...]) + bg1r[...])
        o_ref[...] = _dot(jnp.maximum(pre, 0.0), wg2r[...]) + bg2r[...]

    return pl.pallas_call(
        body,
        out_shape=jax.ShapeDtypeStruct((1, 16), jnp.float32),
    )(nsum128, es128, g, S, We2, be2, Wg1n, Wg1e, Wg1g, bg1, Wg2, bg2)


def _make_sc_edge(N, E, L):
    info = plsc.get_sparse_core_info()
    NC, NS = info.num_cores, info.num_subcores
    NW = NC * NS
    EW = E // NW
    C = 1000
    NCH = EW // C
    CH = 1000
    NJ = N // CH
    mesh = plsc.VectorSubcoreMesh(core_axis_name="c", subcore_axis_name="s")

    @functools.partial(
        pl.kernel,
        mesh=mesh,
        compiler_params=pltpu.CompilerParams(use_tc_tiling_on_sc=False),
        out_type=[
            jax.ShapeDtypeStruct((E // 8, 8 * L), jnp.float32),
            jax.ShapeDtypeStruct((NC * N, L), jnp.float32),
            jax.ShapeDtypeStruct((NC * N, L), jnp.float32),
            jax.ShapeDtypeStruct((NC * N, L), jnp.float32),
            jax.ShapeDtypeStruct((NC * N, L), jnp.float32),
        ],
        scratch_types=[
            pltpu.VMEM((C,), jnp.int32),
            pltpu.VMEM((C,), jnp.int32),
            pltpu.VMEM((C, L), jnp.float32),
            pltpu.VMEM((C, L), jnp.float32),
            pltpu.VMEM((C // 8, 8 * L), jnp.float32),
            pltpu.VMEM((C, L), jnp.float32),
            pltpu.VMEM_SHARED((N, L), jnp.float32),
            pltpu.VMEM_SHARED((N, L), jnp.float32),
            pltpu.VMEM_SHARED((N, L), jnp.float32),
            pltpu.VMEM_SHARED((N, L), jnp.float32),
            pltpu.SemaphoreType.DMA,
            pltpu.SemaphoreType.DMA,
        ],
    )
    def sc_edge(ps_hbm, pr_hbm, elin_hbm, s_hbm, r_hbm,
                h1_out, ss_out, sr_out, cs_out, cr_out,
                idx_s, idx_r, buf_s, buf_r, buf_e, ones_v,
                acc_s, acc_r, acc_cs, acc_cr, sem1, sem2):
        cid = lax.axis_index("c")
        sid = lax.axis_index("s")
        wid = cid * NS + sid

        zero16 = jnp.zeros((16,), jnp.float32)
        one16 = jnp.full((16,), 1.0, jnp.float32)

        def fill(i, _):
            buf_s[i] = zero16
            ones_v[i] = one16
            return 0
        lax.fori_loop(0, C, fill, 0)

        accs = [acc_s, acc_r, acc_cs, acc_cr]
        for a, acc in enumerate(accs):
            for c in range(NJ):
                @pl.when(sid == (a * NJ + c) % NS)
                def _():
                    pltpu.sync_copy(buf_s, acc.at[pl.ds(c * CH, CH)])
        plsc.subcore_barrier()

        ebase = wid * EW
        RB = C // 8
        for k in range(NCH):
            off = ebase + k * C
            off8 = (ebase + k * C) // 8
            pltpu.sync_copy(s_hbm.at[pl.ds(off, C)], idx_s)
            pltpu.sync_copy(r_hbm.at[pl.ds(off, C)], idx_r)
            cp1 = pltpu.async_copy(ps_hbm.at[idx_s], buf_s, sem1)
            cp2 = pltpu.async_copy(pr_hbm.at[idx_r], buf_r, sem2)
            pltpu.sync_copy(elin_hbm.at[pl.ds(off8, RB)], buf_e)
            cp1.wait()
            cp2.wait()

            def addrelu(j, _):
                for m in range(8):
                    i = j * 8 + m
                    v = jnp.maximum(
                        buf_e[j, pl.ds(m * L, L)] + buf_s[i] + buf_r[i], 0.0)
                    buf_e[j, pl.ds(m * L, L)] = v
                    buf_s[i] = v
                return 0
            lax.fori_loop(0, RB, addrelu, 0)

            pltpu.sync_copy(buf_e, h1_out.at[pl.ds(off8, RB)])
            pltpu.sync_copy(buf_s, acc_s.at[idx_s], add=True)
            pltpu.sync_copy(buf_s, acc_r.at[idx_r], add=True)
            pltpu.sync_copy(ones_v, acc_cs.at[idx_s], add=True)
            pltpu.sync_copy(ones_v, acc_cr.at[idx_r], add=True)

        plsc.subcore_barrier()
        outs = [ss_out, sr_out, cs_out, cr_out]
        for a in range(4):
            for c in range(NJ):
                @pl.when(sid == (a * NJ + c) % NS)
                def _(a=a, c=c):
                    pltpu.sync_copy(accs[a].at[pl.ds(c * CH, CH)],
                                    outs[a].at[pl.ds(cid * N + c * CH, CH)])

    return sc_edge


def kernel(nodes, edges, senders, receivers, globals_, We1, be1, We2, be2,
           Wn1, bn1, Wn2, bn2, Wg1, bg1, Wg2, bg2):
    N, DF = nodes.shape
    E, DE = edges.shape
    L = We2.shape[0]

    We1_e = We1[:DE]
    We1_s = We1[DE:DE + DF]
    We1_r = We1[DE + DF:DE + 2 * DF]
    We1_g = We1[DE + 2 * DF:]
    Wn1_n = Wn1[:DF]
    Wn1_s = Wn1[DF:DF + L]
    Wn1_r = Wn1[DF + L:DF + 2 * L]
    Wn1_g = Wn1[DF + 2 * L:]
    Wg1_n = Wg1[:L]
    Wg1_e = Wg1[L:2 * L]
    Wg1_g = Wg1[2 * L:]

    eye8 = jnp.eye(8, dtype=jnp.float32)
    W2BD = jnp.kron(eye8, We2)
    Wn1sBD = jnp.kron(eye8, Wn1_s)
    Wn1rBD = jnp.kron(eye8, Wn1_r)
    Wn2BD = jnp.kron(eye8, Wn2)
    KT = jnp.kron(jnp.ones((1, 8), jnp.float32), jnp.eye(L, dtype=jnp.float32))
    S = jnp.kron(jnp.ones((8, 1), jnp.float32), jnp.eye(L, dtype=jnp.float32))
    be2_128 = jnp.tile(be2.reshape(1, L), (1, 8))
    bn2_128 = jnp.tile(bn2.reshape(1, L), (1, 8))
    be2r = be2.reshape(1, L)

    nodes3 = nodes.reshape(N // 8, 8, DF)
    psb, prb = _node_proj(nodes3, We1_s, We1_r)
    WeBD = jnp.kron(eye8, We1_e)
    elin2 = _edge_linear(edges.reshape(E // 8, 8 * DE), WeBD, globals_,
                         We1_g, be1.reshape(1, L), KT)

    sc_edge = _make_sc_edge(N, E, L)
    h1, ssum, srum, cs16, cr16 = sc_edge(
        psb.reshape(N, L), prb.reshape(N, L), elin2, senders, receivers)

    ne2, es128 = _edge_out(h1, W2BD, be2_128)
    new_edges = ne2.reshape(E, L)

    nnb, nsum128 = _node_block(
        nodes3, ssum.reshape(N // 4, 128), srum.reshape(N // 4, 128),
        cs16.reshape(N // 4, 128), cr16.reshape(N // 4, 128), globals_,
        Wn1_n, Wn1sBD, Wn1rBD, Wn1_g, bn1.reshape(1, L), Wn2BD, bn2_128,
        W2BD, be2_128, KT)
    new_nodes = nnb.reshape(N, L)
    new_globals = _glob_block(nsum128, es128, globals_, S, We2, be2r,
                              Wg1_n, Wg1_e, Wg1_g, bg1.reshape(1, L),
                              Wg2, bg2.reshape(1, L), N, E)
    return new_nodes, new_edges, new_globals
